# baseline (speedup 1.0000x reference)
import jax
import jax.numpy as jnp
from jax.experimental import pallas as pl
from jax.experimental.pallas import tpu as pltpu

N = 8192
D = 64
BM = 256


def _pool_kernel(adj0_ref, adj1_ref, xb_ref, xr_ref, o_ref):
    a0 = adj0_ref[...].astype(jnp.bfloat16)
    a1 = adj1_ref[...].astype(jnp.bfloat16)
    xb = xb_ref[...]
    o_ref[:BM, :] = xr_ref[:BM, :] + jnp.dot(
        a0, xb, preferred_element_type=jnp.float32)
    o_ref[BM:, :] = xr_ref[BM:, :] + jnp.dot(
        a1, xb, preferred_element_type=jnp.float32)


def kernel(x, adj):
    xb = x.astype(jnp.bfloat16)
    return pl.pallas_call(
        _pool_kernel,
        grid=(N // (2 * BM),),
        in_specs=[
            pl.BlockSpec((BM, N), lambda i: (2 * i, 0)),
            pl.BlockSpec((BM, N), lambda i: (2 * i + 1, 0)),
            pl.BlockSpec((N, D), lambda i: (0, 0)),
            pl.BlockSpec((2 * BM, D), lambda i: (i, 0)),
        ],
        out_specs=pl.BlockSpec((2 * BM, D), lambda i: (i, 0)),
        out_shape=jax.ShapeDtypeStruct((N, D), jnp.float32),
        compiler_params=pltpu.CompilerParams(
            dimension_semantics=("arbitrary",),
        ),
    )(adj, adj, xb, x)


# speedup vs baseline: 1.0265x; 1.0265x over previous
import jax
import jax.numpy as jnp
from jax.experimental import pallas as pl
from jax.experimental.pallas import tpu as pltpu

N = 8192
D = 64
BM = 256
NQ = N // 4


def _pool_kernel(a0_ref, a1_ref, a2_ref, a3_ref, xb_ref, xr_ref, o_ref):
    acc = jnp.dot(a0_ref[...].astype(jnp.bfloat16), xb_ref[0 * NQ:1 * NQ, :],
                  preferred_element_type=jnp.float32)
    acc += jnp.dot(a1_ref[...].astype(jnp.bfloat16), xb_ref[1 * NQ:2 * NQ, :],
                   preferred_element_type=jnp.float32)
    acc += jnp.dot(a2_ref[...].astype(jnp.bfloat16), xb_ref[2 * NQ:3 * NQ, :],
                   preferred_element_type=jnp.float32)
    acc += jnp.dot(a3_ref[...].astype(jnp.bfloat16), xb_ref[3 * NQ:4 * NQ, :],
                   preferred_element_type=jnp.float32)
    o_ref[...] = xr_ref[...] + acc


def kernel(x, adj):
    xb = x.astype(jnp.bfloat16)
    return pl.pallas_call(
        _pool_kernel,
        grid=(N // BM,),
        in_specs=[
            pl.BlockSpec((BM, NQ), lambda i: (i, 0)),
            pl.BlockSpec((BM, NQ), lambda i: (i, 1)),
            pl.BlockSpec((BM, NQ), lambda i: (i, 2)),
            pl.BlockSpec((BM, NQ), lambda i: (i, 3)),
            pl.BlockSpec((N, D), lambda i: (0, 0)),
            pl.BlockSpec((BM, D), lambda i: (i, 0)),
        ],
        out_specs=pl.BlockSpec((BM, D), lambda i: (i, 0)),
        out_shape=jax.ShapeDtypeStruct((N, D), jnp.float32),
        compiler_params=pltpu.CompilerParams(
            dimension_semantics=("arbitrary",),
        ),
    )(adj, adj, adj, adj, xb, x)
